# baseline (speedup 1.0000x reference)
import functools

import jax
import jax.numpy as jnp
from jax import lax
from jax.experimental import pallas as pl
from jax.experimental.pallas import tpu as pltpu
from jax.experimental.pallas import tpu_sc as plsc

H = 128
N0, N1 = 10000, 30000
M0P, M1P = 10240, 30080
HALF0, HALF1 = 5120, 15040
EP0, EP1, EPI = 320000, 320000, 61440
BM = 320
BMC = 2560
KB = 80


def _silu(t):
    return t * jax.nn.sigmoid(t)


def _linear_body(x_ref, w_ref, b_ref, o_ref):
    o_ref[...] = jnp.dot(x_ref[...], w_ref[...],
                         preferred_element_type=jnp.float32) + b_ref[...]


def _linear(x, w, b):
    m, k = x.shape
    n = w.shape[1]
    return pl.pallas_call(
        _linear_body,
        grid=(m // BM,),
        in_specs=[pl.BlockSpec((BM, k), lambda i: (i, 0)),
                  pl.BlockSpec((k, n), lambda i: (0, 0)),
                  pl.BlockSpec((1, n), lambda i: (0, 0))],
        out_specs=pl.BlockSpec((BM, n), lambda i: (i, 0)),
        out_shape=jax.ShapeDtypeStruct((m, n), jnp.float32),
    )(x, w, b.reshape(1, n))


def _ab_body(x_ref, wa_ref, wb_ref, a_ref, b_ref):
    x = x_ref[...]
    a_ref[...] = jnp.dot(x, wa_ref[...], preferred_element_type=jnp.float32)
    b_ref[...] = jnp.dot(x, wb_ref[...], preferred_element_type=jnp.float32)


def _ab(x, wa, wb):
    m, k = x.shape
    na, nb = wa.shape[1], wb.shape[1]
    return pl.pallas_call(
        _ab_body,
        grid=(m // BM,),
        in_specs=[pl.BlockSpec((BM, k), lambda i: (i, 0)),
                  pl.BlockSpec((k, na), lambda i: (0, 0)),
                  pl.BlockSpec((k, nb), lambda i: (0, 0))],
        out_specs=[pl.BlockSpec((BM, na), lambda i: (i, 0)),
                   pl.BlockSpec((BM, nb), lambda i: (i, 0))],
        out_shape=[jax.ShapeDtypeStruct((m, na), jnp.float32),
                   jax.ShapeDtypeStruct((m, nb), jnp.float32)],
    )(x, wa, wb)


def _mm_body(x_ref, w_ref, o_ref):
    o_ref[...] = jnp.dot(x_ref[...], w_ref[...],
                         preferred_element_type=jnp.float32)


def _mm(x, w):
    m, k = x.shape
    n = w.shape[1]
    return pl.pallas_call(
        _mm_body,
        grid=(m // BM,),
        in_specs=[pl.BlockSpec((BM, k), lambda i: (i, 0)),
                  pl.BlockSpec((k, n), lambda i: (0, 0))],
        out_specs=pl.BlockSpec((BM, n), lambda i: (i, 0)),
        out_shape=jax.ShapeDtypeStruct((m, n), jnp.float32),
    )(x, w)


def _edgec_body(inv_ref, w_ref, b_ref, o_ref):
    inv = inv_ref[...]
    o_ref[...] = (b_ref[...]
                  + inv[:, 0:1] * w_ref[0:1, :]
                  + inv[:, 1:2] * w_ref[1:2, :]
                  + inv[:, 2:3] * w_ref[2:3, :])


def _edgec(inv, w, b):
    m = inv.shape[0]
    n = w.shape[1]
    return pl.pallas_call(
        _edgec_body,
        grid=(m // BMC,),
        in_specs=[pl.BlockSpec((BMC, 3), lambda i: (i, 0)),
                  pl.BlockSpec((3, n), lambda i: (0, 0)),
                  pl.BlockSpec((1, n), lambda i: (0, 0))],
        out_specs=pl.BlockSpec((BMC, n), lambda i: (i, 0)),
        out_shape=jax.ShapeDtypeStruct((m, n), jnp.float32),
    )(inv, w, b.reshape(1, n))


def _update_body(x_ref, w_ref, b_ref, r_ref, o_ref):
    t = jnp.dot(x_ref[...], w_ref[...],
                preferred_element_type=jnp.float32) + b_ref[...]
    o_ref[...] = r_ref[...] + _silu(t)


def _update(x, w, b, res):
    m, k = x.shape
    return pl.pallas_call(
        _update_body,
        grid=(m // BM,),
        in_specs=[pl.BlockSpec((BM, k), lambda i: (i, 0)),
                  pl.BlockSpec((k, H), lambda i: (0, 0)),
                  pl.BlockSpec((1, H), lambda i: (0, 0)),
                  pl.BlockSpec((BM, H), lambda i: (i, 0))],
        out_specs=pl.BlockSpec((BM, H), lambda i: (i, 0)),
        out_shape=jax.ShapeDtypeStruct((m, H), jnp.float32),
    )(x, w, b.reshape(1, H), res)


def _pool_body(x_ref, w1_ref, b1_ref, w2_ref, b2_ref, o_ref, *, n_real):
    i = pl.program_id(0)
    t = jnp.dot(x_ref[...], w1_ref[...],
                preferred_element_type=jnp.float32) + b1_ref[...]
    p = jnp.dot(_silu(t), w2_ref[...],
                preferred_element_type=jnp.float32) + b2_ref[...]
    rows = i * BM + lax.broadcasted_iota(jnp.int32, (BM, 1), 0)
    p = jnp.where(rows < n_real, p, 0.0)
    s = jnp.sum(p, axis=0, keepdims=True)

    @pl.when(i == 0)
    def _():
        o_ref[...] = jnp.zeros_like(o_ref)

    o_ref[0:1, :] = o_ref[0:1, :] + s


def _pool(x, w1, b1, w2, b2, n_real):
    m = x.shape[0]
    return pl.pallas_call(
        functools.partial(_pool_body, n_real=n_real),
        grid=(m // BM,),
        in_specs=[pl.BlockSpec((BM, H), lambda i: (i, 0)),
                  pl.BlockSpec((H, H), lambda i: (0, 0)),
                  pl.BlockSpec((1, H), lambda i: (0, 0)),
                  pl.BlockSpec((H, H), lambda i: (0, 0)),
                  pl.BlockSpec((1, H), lambda i: (0, 0))],
        out_specs=pl.BlockSpec((8, H), lambda i: (0, 0)),
        out_shape=jax.ShapeDtypeStruct((8, H), jnp.float32),
    )(x, w1, b1.reshape(1, H), w2, b2.reshape(1, H))


def _head_body(g0_ref, g1_ref, w1a_ref, w1b_ref, b1_ref, w2_ref, b2_ref, o_ref):
    t = (jnp.dot(g0_ref[...], w1a_ref[...], preferred_element_type=jnp.float32)
         + jnp.dot(g1_ref[...], w1b_ref[...], preferred_element_type=jnp.float32)
         + b1_ref[...])
    o_ref[...] = jnp.dot(_silu(t), w2_ref[...],
                         preferred_element_type=jnp.float32) + b2_ref[...]


def _head(g0, g1, w1a, w1b, b1, w2, b2):
    return pl.pallas_call(
        _head_body,
        out_shape=jax.ShapeDtypeStruct((8, H), jnp.float32),
    )(g0, g1, w1a, w1b, b1.reshape(1, H), w2, b2.reshape(1, H))


@functools.lru_cache(maxsize=None)
def _make_seg(qsize, ep, cols, dpad, nq=1):
    fch = 40
    per_tile = ep // 16
    DR = 6 * KB
    CAP = DR + 16
    CH = 2000 if per_tile % 2000 == 0 else 384
    assert per_tile % CH == 0 and CH % 16 == 0
    cap_out = per_tile + DR
    n_chunks = qsize // fch
    per_sub = -(-n_chunks // 16)
    acc_rows = qsize + 8
    n_out = 2 * nq * qsize
    mesh = plsc.VectorSubcoreMesh(core_axis_name="c", subcore_axis_name="s",
                                  num_cores=2, num_subcores=16)
    iota16 = lambda: lax.broadcasted_iota(jnp.int32, (16,), 0)

    vset = lambda: [pltpu.VMEM((KB,), jnp.int32),
                    pltpu.VMEM((KB,), jnp.int32),
                    pltpu.VMEM((KB,), jnp.int32),
                    pltpu.VMEM((KB,), jnp.int32),
                    pltpu.VMEM((KB, cols), jnp.float32),
                    pltpu.VMEM((KB, cols), jnp.float32),
                    pltpu.VMEM((KB, cols), jnp.float32),
                    pltpu.SemaphoreType.DMA,
                    pltpu.SemaphoreType.DMA,
                    pltpu.SemaphoreType.DMA]

    @functools.partial(
        pl.kernel,
        out_type=[jax.ShapeDtypeStruct((n_out, cols), jnp.float32),
                  jax.ShapeDtypeStruct((32 * cap_out,), jnp.int32),
                  jax.ShapeDtypeStruct((32 * cap_out,), jnp.int32),
                  jax.ShapeDtypeStruct((32 * cap_out,), jnp.int32)],
        mesh=mesh,
        compiler_params=pltpu.CompilerParams(needs_layout_passes=False),
        scratch_types=[*vset(), *vset(),
                       pltpu.VMEM((CH,), jnp.int32),
                       pltpu.VMEM((CH,), jnp.int32),
                       pltpu.VMEM((CAP,), jnp.int32),
                       pltpu.VMEM((CAP,), jnp.int32),
                       pltpu.VMEM((CAP,), jnp.int32),
                       pltpu.VMEM_SHARED((acc_rows, cols), jnp.float32)],
    )
    def seg(a_hbm, b_hbm, c_hbm, src_hbm, dst_hbm,
            out_hbm, cs_hbm, cd_hbm, ce_hbm,
            si0, di0, ei0, sl0, av0, bv0, cv0, sa0, sb0, sc0,
            si1, di1, ei1, sl1, av1, bv1, cv1, sa1, sb1, sc1,
            p1s, p1d, csb, cdb, ceb, acc):
        cid = lax.axis_index("c")
        sid = lax.axis_index("s")
        tile_base = sid * per_tile
        obase = (cid * 16 + sid) * cap_out
        sets = ((si0, di0, ei0, sl0, av0, bv0, cv0, sa0, sb0, sc0),
                (si1, di1, ei1, sl1, av1, bv1, cv1, sa1, sb1, sc1))

        def _zrow(i, _):
            for j in range(cols // 16):
                cv0[i, pl.ds(j * 16, 16)] = jnp.zeros((16,), jnp.float32)
            return 0
        lax.fori_loop(0, fch, _zrow, 0)

        def _drain(carry):
            ptr, tot = carry
            off = pl.multiple_of(obase + tot, 8)
            pltpu.sync_copy(csb.at[pl.ds(0, DR)], cs_hbm.at[pl.ds(off, DR)])
            pltpu.sync_copy(cdb.at[pl.ds(0, DR)], cd_hbm.at[pl.ds(off, DR)])
            pltpu.sync_copy(ceb.at[pl.ds(0, DR)], ce_hbm.at[pl.ds(off, DR)])
            csb[pl.ds(0, 16)] = csb[pl.ds(DR, 16)]
            cdb[pl.ds(0, 16)] = cdb[pl.ds(DR, 16)]
            ceb[pl.ds(0, 16)] = ceb[pl.ds(DR, 16)]
            return ptr - DR, tot + DR

        def _filter(base_row):
            def _chunk(t, carry):
                ebase = tile_base + t * CH
                pltpu.sync_copy(src_hbm.at[pl.ds(ebase, CH)], p1s)
                pltpu.sync_copy(dst_hbm.at[pl.ds(ebase, CH)], p1d)

                def _grp(j, c2):
                    ptr, tot = c2
                    s16 = p1s[pl.ds(j * 16, 16)]
                    d16 = p1d[pl.ds(j * 16, 16)]
                    e16 = ebase + j * 16 + iota16()
                    loc = d16 - base_row
                    ok = (loc >= 0) & (loc < qsize)
                    oki = ok.astype(jnp.int32)
                    cum = plsc.cumsum(oki)
                    pos = ptr + cum - oki
                    plsc.store_scatter(csb, [pos], s16, mask=ok)
                    plsc.store_scatter(cdb, [pos], d16, mask=ok)
                    plsc.store_scatter(ceb, [pos], e16, mask=ok)
                    ptr = ptr + jnp.sum(oki)
                    return lax.cond(ptr >= DR, _drain, lambda c: c, (ptr, tot))
                return lax.fori_loop(0, CH // 16, _grp, carry)

            ptr, tot = lax.fori_loop(0, per_tile // CH, _chunk, (0, 0))
            for j in range(DR // 16):
                lane = j * 16 + iota16()
                m = lane >= ptr
                sl = pl.ds(j * 16, 16)
                csb[sl] = jnp.where(m, 0, csb[sl])
                cdb[sl] = jnp.where(m, dpad, cdb[sl])
                ceb[sl] = jnp.where(m, 0, ceb[sl])
            _, tot = _drain((ptr, tot))
            return tot

        def _stage(bi, base_row, st):
            sidx, didx, eidx, slot, av, bv, cv, sa, sb, sc = st
            off = pl.multiple_of(obase + bi * KB, 8)
            pltpu.sync_copy(cs_hbm.at[pl.ds(off, KB)], sidx)
            pltpu.sync_copy(cd_hbm.at[pl.ds(off, KB)], didx)
            pltpu.sync_copy(ce_hbm.at[pl.ds(off, KB)], eidx)
            pltpu.async_copy(a_hbm.at[sidx], av, sa)
            pltpu.async_copy(b_hbm.at[didx], bv, sb)
            pltpu.async_copy(c_hbm.at[eidx], cv, sc)
            for j in range(KB // 16):
                d = didx[pl.ds(j * 16, 16)]
                loc = d - base_row
                ok = (loc >= 0) & (loc < qsize)
                slot[pl.ds(j * 16, 16)] = jnp.where(ok, loc, qsize)

        def _proc(st):
            sidx, didx, eidx, slot, av, bv, cv, sa, sb, sc = st
            pltpu.make_async_copy(a_hbm.at[sidx], av, sa).wait()
            pltpu.make_async_copy(b_hbm.at[didx], bv, sb).wait()
            pltpu.make_async_copy(c_hbm.at[eidx], cv, sc).wait()

            def _edge(i, _):
                for j in range(cols // 16):
                    sl = pl.ds(j * 16, 16)
                    t = av[i, sl] + bv[i, sl] + cv[i, sl]
                    av[i, sl] = t / (1.0 + jnp.exp(-t))
                return 0
            lax.fori_loop(0, KB, _edge, 0)
            pltpu.sync_copy(av, acc.at[slot], add=True)

        for p in range(nq):
            base_row = (2 * p + cid) * qsize

            def _zacc(t, _):
                ch = t * 16 + sid

                @pl.when(ch < n_chunks)
                def _():
                    pltpu.sync_copy(cv0.at[pl.ds(0, fch)],
                                    acc.at[pl.ds(ch * fch, fch)])
                return 0
            lax.fori_loop(0, per_sub, _zacc, 0)

            @pl.when(sid == 0)
            def _():
                pltpu.sync_copy(cv0.at[pl.ds(0, 8)], acc.at[pl.ds(qsize, 8)])

            plsc.subcore_barrier()

            total = _filter(base_row)
            n2 = total // (2 * KB)

            _stage(0, base_row, sets[0])

            def _gbody(g, _):
                _stage(2 * g + 1, base_row, sets[1])
                _proc(sets[0])

                @pl.when(g + 1 < n2)
                def _():
                    _stage(2 * g + 2, base_row, sets[0])
                _proc(sets[1])
                return 0
            lax.fori_loop(0, n2, _gbody, 0)

            plsc.subcore_barrier()

            for t in range(per_sub):
                ch = t * 16 + sid

                @pl.when(ch < n_chunks)
                def _():
                    r0 = ch * fch
                    pltpu.sync_copy(acc.at[pl.ds(r0, fch)], cv1.at[pl.ds(0, fch)])
                    pltpu.sync_copy(cv1.at[pl.ds(0, fch)],
                                    out_hbm.at[pl.ds(base_row + r0, fch)])

            if p + 1 < nq:
                plsc.subcore_barrier()
                lax.fori_loop(0, fch, _zrow, 0)

    return seg


def _pad_rows(x, m):
    return jnp.pad(x, ((0, m - x.shape[0]), (0, 0)))


def _pad_idx(idx, ep, fill):
    return jnp.pad(idx, (0, ep - idx.shape[0]), constant_values=fill)


def kernel(x0, x1, adj0, adj1, inc1, inv0, inv1, inv_inc1, W_emb, b_emb,
           Wm_adj0, bm_adj0, Wm_adj1, bm_adj1, Wm_inc1, bm_inc1, Wu0, bu0,
           Wu1, bu1, Wpre0, bpre0, Wpre1, bpre1, Wpost1, bpost1, Wpost2,
           bpost2):
    x0p = _pad_rows(x0, M0P)
    x1p = _pad_rows(x1, M1P)
    s0 = _pad_idx(adj0[0], EP0, 0)
    d0 = _pad_idx(adj0[1], EP0, M0P - 1)
    s1 = _pad_idx(adj1[0], EP1, 0)
    d1 = _pad_idx(adj1[1], EP1, M1P - 1)
    si = _pad_idx(inc1[0], EPI, 0)
    di = _pad_idx(inc1[1], EPI, M0P)
    inv0p = _pad_rows(inv0, EP0)
    inv1p = _pad_rows(inv1, EP1)
    invip = _pad_rows(inv_inc1, EPI)

    h0 = _linear(x0p, W_emb, b_emb)
    h1 = _linear(x1p, W_emb, b_emb)

    for l in range(2):
        A0, B0 = _ab(h0, Wm_adj0[l, :H], Wm_adj0[l, H:2 * H])
        A1, B1 = _ab(h1, Wm_adj1[l, :H], Wm_adj1[l, H:2 * H])
        Ai = _mm(h0, Wm_inc1[l, :H])
        Bi = _mm(h1, Wm_inc1[l, H:2 * H])
        C0 = _edgec(inv0p, Wm_adj0[l, 2 * H:], bm_adj0[l])
        C1 = _edgec(inv1p, Wm_adj1[l, 2 * H:], bm_adj1[l])
        Ci = _edgec(invip, Wm_inc1[l, 2 * H:], bm_inc1[l])

        m0 = _make_seg(HALF0, EP0, H, M0P - 1)(A0, B0, C0, s0, d0)[0]
        m1 = _make_seg(M1P // 4, EP1, H, M1P - 1, nq=2)(A1, B1, C1, s1, d1)[0]
        mi = _make_seg(HALF0, EPI, H, M0P)(Ai, Bi, Ci, si, di)[0]
        mip = jnp.pad(mi, ((0, M1P - M0P), (0, 0)))

        h0 = _update(jnp.concatenate([h0, m0], axis=1), Wu0[l], bu0[l], h0)
        h1 = _update(jnp.concatenate([h1, m1, mip], axis=1), Wu1[l], bu1[l], h1)

    g0 = _pool(h0, Wpre0[0], bpre0[0], Wpre0[1], bpre0[1], N0)
    g1 = _pool(h1, Wpre1[0], bpre1[0], Wpre1[1], bpre1[1], N1)
    out8 = _head(g0, g1, Wpost1[:H], Wpost1[H:], bpost1, Wpost2, bpost2)
    return out8[0]


# speedup vs baseline: 1.2707x; 1.2707x over previous
import functools

import jax
import jax.numpy as jnp
from jax import lax
from jax.experimental import pallas as pl
from jax.experimental.pallas import tpu as pltpu
from jax.experimental.pallas import tpu_sc as plsc

H = 128
N0, N1 = 10000, 30000
M0P, M1P = 10240, 30080
HALF0, HALF1 = 5120, 15040
EP0, EP1, EPI = 320000, 320000, 61440
BM = 320
BMC = 2560
KB = 80


def _silu(t):
    return t * jax.nn.sigmoid(t)


def _linear_body(x_ref, w_ref, b_ref, o_ref):
    o_ref[...] = jnp.dot(x_ref[...], w_ref[...],
                         preferred_element_type=jnp.float32) + b_ref[...]


def _linear(x, w, b):
    m, k = x.shape
    n = w.shape[1]
    return pl.pallas_call(
        _linear_body,
        grid=(m // BM,),
        in_specs=[pl.BlockSpec((BM, k), lambda i: (i, 0)),
                  pl.BlockSpec((k, n), lambda i: (0, 0)),
                  pl.BlockSpec((1, n), lambda i: (0, 0))],
        out_specs=pl.BlockSpec((BM, n), lambda i: (i, 0)),
        out_shape=jax.ShapeDtypeStruct((m, n), jnp.float32),
    )(x, w, b.reshape(1, n))


def _ab_body(x_ref, wa_ref, wb_ref, a_ref, b_ref):
    x = x_ref[...]
    a_ref[...] = jnp.dot(x, wa_ref[...], preferred_element_type=jnp.float32)
    b_ref[...] = jnp.dot(x, wb_ref[...], preferred_element_type=jnp.float32)


def _ab(x, wa, wb):
    m, k = x.shape
    na, nb = wa.shape[1], wb.shape[1]
    return pl.pallas_call(
        _ab_body,
        grid=(m // BM,),
        in_specs=[pl.BlockSpec((BM, k), lambda i: (i, 0)),
                  pl.BlockSpec((k, na), lambda i: (0, 0)),
                  pl.BlockSpec((k, nb), lambda i: (0, 0))],
        out_specs=[pl.BlockSpec((BM, na), lambda i: (i, 0)),
                   pl.BlockSpec((BM, nb), lambda i: (i, 0))],
        out_shape=[jax.ShapeDtypeStruct((m, na), jnp.float32),
                   jax.ShapeDtypeStruct((m, nb), jnp.float32)],
    )(x, wa, wb)


def _mm_body(x_ref, w_ref, o_ref):
    o_ref[...] = jnp.dot(x_ref[...], w_ref[...],
                         preferred_element_type=jnp.float32)


def _mm(x, w):
    m, k = x.shape
    n = w.shape[1]
    return pl.pallas_call(
        _mm_body,
        grid=(m // BM,),
        in_specs=[pl.BlockSpec((BM, k), lambda i: (i, 0)),
                  pl.BlockSpec((k, n), lambda i: (0, 0))],
        out_specs=pl.BlockSpec((BM, n), lambda i: (i, 0)),
        out_shape=jax.ShapeDtypeStruct((m, n), jnp.float32),
    )(x, w)


def _edgec_body(inv_ref, w_ref, b_ref, o_ref):
    inv = inv_ref[...]
    o_ref[...] = (b_ref[...]
                  + inv[:, 0:1] * w_ref[0:1, :]
                  + inv[:, 1:2] * w_ref[1:2, :]
                  + inv[:, 2:3] * w_ref[2:3, :])


def _edgec(inv, w, b):
    m = inv.shape[0]
    n = w.shape[1]
    return pl.pallas_call(
        _edgec_body,
        grid=(m // BMC,),
        in_specs=[pl.BlockSpec((BMC, 3), lambda i: (i, 0)),
                  pl.BlockSpec((3, n), lambda i: (0, 0)),
                  pl.BlockSpec((1, n), lambda i: (0, 0))],
        out_specs=pl.BlockSpec((BMC, n), lambda i: (i, 0)),
        out_shape=jax.ShapeDtypeStruct((m, n), jnp.float32),
    )(inv, w, b.reshape(1, n))


def _update_body(x_ref, w_ref, b_ref, r_ref, o_ref):
    t = jnp.dot(x_ref[...], w_ref[...],
                preferred_element_type=jnp.float32) + b_ref[...]
    o_ref[...] = r_ref[...] + _silu(t)


def _update(x, w, b, res):
    m, k = x.shape
    return pl.pallas_call(
        _update_body,
        grid=(m // BM,),
        in_specs=[pl.BlockSpec((BM, k), lambda i: (i, 0)),
                  pl.BlockSpec((k, H), lambda i: (0, 0)),
                  pl.BlockSpec((1, H), lambda i: (0, 0)),
                  pl.BlockSpec((BM, H), lambda i: (i, 0))],
        out_specs=pl.BlockSpec((BM, H), lambda i: (i, 0)),
        out_shape=jax.ShapeDtypeStruct((m, H), jnp.float32),
    )(x, w, b.reshape(1, H), res)


def _pool_body(x_ref, w1_ref, b1_ref, w2_ref, b2_ref, o_ref, *, n_real):
    i = pl.program_id(0)
    t = jnp.dot(x_ref[...], w1_ref[...],
                preferred_element_type=jnp.float32) + b1_ref[...]
    p = jnp.dot(_silu(t), w2_ref[...],
                preferred_element_type=jnp.float32) + b2_ref[...]
    rows = i * BM + lax.broadcasted_iota(jnp.int32, (BM, 1), 0)
    p = jnp.where(rows < n_real, p, 0.0)
    s = jnp.sum(p, axis=0, keepdims=True)

    @pl.when(i == 0)
    def _():
        o_ref[...] = jnp.zeros_like(o_ref)

    o_ref[0:1, :] = o_ref[0:1, :] + s


def _pool(x, w1, b1, w2, b2, n_real):
    m = x.shape[0]
    return pl.pallas_call(
        functools.partial(_pool_body, n_real=n_real),
        grid=(m // BM,),
        in_specs=[pl.BlockSpec((BM, H), lambda i: (i, 0)),
                  pl.BlockSpec((H, H), lambda i: (0, 0)),
                  pl.BlockSpec((1, H), lambda i: (0, 0)),
                  pl.BlockSpec((H, H), lambda i: (0, 0)),
                  pl.BlockSpec((1, H), lambda i: (0, 0))],
        out_specs=pl.BlockSpec((8, H), lambda i: (0, 0)),
        out_shape=jax.ShapeDtypeStruct((8, H), jnp.float32),
    )(x, w1, b1.reshape(1, H), w2, b2.reshape(1, H))


def _head_body(g0_ref, g1_ref, w1a_ref, w1b_ref, b1_ref, w2_ref, b2_ref, o_ref):
    t = (jnp.dot(g0_ref[...], w1a_ref[...], preferred_element_type=jnp.float32)
         + jnp.dot(g1_ref[...], w1b_ref[...], preferred_element_type=jnp.float32)
         + b1_ref[...])
    o_ref[...] = jnp.dot(_silu(t), w2_ref[...],
                         preferred_element_type=jnp.float32) + b2_ref[...]


def _head(g0, g1, w1a, w1b, b1, w2, b2):
    return pl.pallas_call(
        _head_body,
        out_shape=jax.ShapeDtypeStruct((8, H), jnp.float32),
    )(g0, g1, w1a, w1b, b1.reshape(1, H), w2, b2.reshape(1, H))


@functools.lru_cache(maxsize=None)
def _make_seg(qsize, ep, cols, dpad, nq=1):
    fch = 40
    per_tile = ep // 16
    DR = 4 * KB
    CAP = DR + 16
    CH = 400 if per_tile % 400 == 0 else 384
    assert per_tile % CH == 0 and CH % 16 == 0
    cap_out = per_tile + DR
    n_chunks = qsize // fch
    per_sub = -(-n_chunks // 16)
    acc_rows = qsize + 8
    n_out = 2 * nq * qsize
    mesh = plsc.VectorSubcoreMesh(core_axis_name="c", subcore_axis_name="s",
                                  num_cores=2, num_subcores=16)
    iota16 = lambda: lax.broadcasted_iota(jnp.int32, (16,), 0)

    vset = lambda: [pltpu.VMEM((KB,), jnp.int32),
                    pltpu.VMEM((KB,), jnp.int32),
                    pltpu.VMEM((KB,), jnp.int32),
                    pltpu.VMEM((KB,), jnp.int32),
                    pltpu.VMEM((KB, cols), jnp.float32),
                    pltpu.VMEM((KB, cols), jnp.float32),
                    pltpu.VMEM((KB, cols), jnp.float32),
                    pltpu.SemaphoreType.DMA,
                    pltpu.SemaphoreType.DMA,
                    pltpu.SemaphoreType.DMA]

    @functools.partial(
        pl.kernel,
        out_type=[jax.ShapeDtypeStruct((n_out, cols), jnp.float32),
                  jax.ShapeDtypeStruct((32 * cap_out,), jnp.int32),
                  jax.ShapeDtypeStruct((32 * cap_out,), jnp.int32),
                  jax.ShapeDtypeStruct((32 * cap_out,), jnp.int32)],
        mesh=mesh,
        compiler_params=pltpu.CompilerParams(needs_layout_passes=False),
        scratch_types=[*vset(), *vset(),
                       pltpu.VMEM((CH,), jnp.int32),
                       pltpu.VMEM((CH,), jnp.int32),
                       pltpu.VMEM((CAP,), jnp.int32),
                       pltpu.VMEM((CAP,), jnp.int32),
                       pltpu.VMEM((CAP,), jnp.int32),
                       pltpu.VMEM_SHARED((acc_rows, cols), jnp.float32)],
    )
    def seg(a_hbm, b_hbm, c_hbm, src_hbm, dst_hbm,
            out_hbm, cs_hbm, cd_hbm, ce_hbm,
            si0, di0, ei0, sl0, av0, bv0, cv0, sa0, sb0, sc0,
            si1, di1, ei1, sl1, av1, bv1, cv1, sa1, sb1, sc1,
            p1s, p1d, csb, cdb, ceb, acc):
        cid = lax.axis_index("c")
        sid = lax.axis_index("s")
        tile_base = sid * per_tile
        obase = (cid * 16 + sid) * cap_out
        sets = ((si0, di0, ei0, sl0, av0, bv0, cv0, sa0, sb0, sc0),
                (si1, di1, ei1, sl1, av1, bv1, cv1, sa1, sb1, sc1))

        def _zrow(i, _):
            for j in range(cols // 16):
                cv0[i, pl.ds(j * 16, 16)] = jnp.zeros((16,), jnp.float32)
            return 0
        lax.fori_loop(0, fch, _zrow, 0)

        def _drain(carry):
            ptr, tot = carry
            off = pl.multiple_of(obase + tot, 8)
            pltpu.sync_copy(csb.at[pl.ds(0, DR)], cs_hbm.at[pl.ds(off, DR)])
            pltpu.sync_copy(cdb.at[pl.ds(0, DR)], cd_hbm.at[pl.ds(off, DR)])
            pltpu.sync_copy(ceb.at[pl.ds(0, DR)], ce_hbm.at[pl.ds(off, DR)])
            csb[pl.ds(0, 16)] = csb[pl.ds(DR, 16)]
            cdb[pl.ds(0, 16)] = cdb[pl.ds(DR, 16)]
            ceb[pl.ds(0, 16)] = ceb[pl.ds(DR, 16)]
            return ptr - DR, tot + DR

        def _filter(base_row):
            def _chunk(t, carry):
                ebase = tile_base + t * CH
                pltpu.sync_copy(src_hbm.at[pl.ds(ebase, CH)], p1s)
                pltpu.sync_copy(dst_hbm.at[pl.ds(ebase, CH)], p1d)

                def _grp(j, c2):
                    ptr, tot = c2
                    s16 = p1s[pl.ds(j * 16, 16)]
                    d16 = p1d[pl.ds(j * 16, 16)]
                    e16 = ebase + j * 16 + iota16()
                    loc = d16 - base_row
                    ok = (loc >= 0) & (loc < qsize)
                    oki = ok.astype(jnp.int32)
                    cum = plsc.cumsum(oki)
                    pos = ptr + cum - oki
                    plsc.store_scatter(csb, [pos], s16, mask=ok)
                    plsc.store_scatter(cdb, [pos], d16, mask=ok)
                    plsc.store_scatter(ceb, [pos], e16, mask=ok)
                    ptr = ptr + jnp.sum(oki)
                    return lax.cond(ptr >= DR, _drain, lambda c: c, (ptr, tot))
                return lax.fori_loop(0, CH // 16, _grp, carry)

            ptr, tot = lax.fori_loop(0, per_tile // CH, _chunk, (0, 0))
            for j in range(DR // 16):
                lane = j * 16 + iota16()
                m = lane >= ptr
                sl = pl.ds(j * 16, 16)
                csb[sl] = jnp.where(m, 0, csb[sl])
                cdb[sl] = jnp.where(m, dpad, cdb[sl])
                ceb[sl] = jnp.where(m, 0, ceb[sl])
            _, tot = _drain((ptr, tot))
            return tot

        def _stage(bi, base_row, st):
            sidx, didx, eidx, slot, av, bv, cv, sa, sb, sc = st
            off = pl.multiple_of(obase + bi * KB, 8)
            pltpu.sync_copy(cs_hbm.at[pl.ds(off, KB)], sidx)
            pltpu.sync_copy(cd_hbm.at[pl.ds(off, KB)], didx)
            pltpu.sync_copy(ce_hbm.at[pl.ds(off, KB)], eidx)
            pltpu.async_copy(a_hbm.at[sidx], av, sa)
            pltpu.async_copy(b_hbm.at[didx], bv, sb)
            pltpu.async_copy(c_hbm.at[eidx], cv, sc)
            for j in range(KB // 16):
                d = didx[pl.ds(j * 16, 16)]
                loc = d - base_row
                ok = (loc >= 0) & (loc < qsize)
                slot[pl.ds(j * 16, 16)] = jnp.where(ok, loc, qsize)

        def _proc(st):
            sidx, didx, eidx, slot, av, bv, cv, sa, sb, sc = st
            pltpu.make_async_copy(a_hbm.at[sidx], av, sa).wait()
            pltpu.make_async_copy(b_hbm.at[didx], bv, sb).wait()
            pltpu.make_async_copy(c_hbm.at[eidx], cv, sc).wait()

            def _edge(i, _):
                for j in range(cols // 16):
                    sl = pl.ds(j * 16, 16)
                    t = av[i, sl] + bv[i, sl] + cv[i, sl]
                    av[i, sl] = t / (1.0 + jnp.exp(-t))
                return 0
            lax.fori_loop(0, KB, _edge, 0)
            pltpu.sync_copy(av, acc.at[slot], add=True)

        for p in range(nq):
            base_row = (2 * p + cid) * qsize

            def _zacc(t, _):
                ch = t * 16 + sid

                @pl.when(ch < n_chunks)
                def _():
                    pltpu.sync_copy(cv0.at[pl.ds(0, fch)],
                                    acc.at[pl.ds(ch * fch, fch)])
                return 0
            lax.fori_loop(0, per_sub, _zacc, 0)

            @pl.when(sid == 0)
            def _():
                pltpu.sync_copy(cv0.at[pl.ds(0, 8)], acc.at[pl.ds(qsize, 8)])

            plsc.subcore_barrier()

            total = _filter(base_row)
            n2 = total // (2 * KB)

            _stage(0, base_row, sets[0])

            def _gbody(g, _):
                _stage(2 * g + 1, base_row, sets[1])
                _proc(sets[0])

                @pl.when(g + 1 < n2)
                def _():
                    _stage(2 * g + 2, base_row, sets[0])
                _proc(sets[1])
                return 0
            lax.fori_loop(0, n2, _gbody, 0)

            plsc.subcore_barrier()

            for t in range(per_sub):
                ch = t * 16 + sid

                @pl.when(ch < n_chunks)
                def _():
                    r0 = ch * fch
                    pltpu.sync_copy(acc.at[pl.ds(r0, fch)], cv1.at[pl.ds(0, fch)])
                    pltpu.sync_copy(cv1.at[pl.ds(0, fch)],
                                    out_hbm.at[pl.ds(base_row + r0, fch)])

            if p + 1 < nq:
                plsc.subcore_barrier()
                lax.fori_loop(0, fch, _zrow, 0)

    return seg


def _pad_rows(x, m):
    return jnp.pad(x, ((0, m - x.shape[0]), (0, 0)))


def _pad_idx(idx, ep, fill):
    return jnp.pad(idx, (0, ep - idx.shape[0]), constant_values=fill)


def kernel(x0, x1, adj0, adj1, inc1, inv0, inv1, inv_inc1, W_emb, b_emb,
           Wm_adj0, bm_adj0, Wm_adj1, bm_adj1, Wm_inc1, bm_inc1, Wu0, bu0,
           Wu1, bu1, Wpre0, bpre0, Wpre1, bpre1, Wpost1, bpost1, Wpost2,
           bpost2):
    x0p = _pad_rows(x0, M0P)
    x1p = _pad_rows(x1, M1P)
    s0 = _pad_idx(adj0[0], EP0, 0)
    d0 = _pad_idx(adj0[1], EP0, M0P - 1)
    s1 = _pad_idx(adj1[0], EP1, 0)
    d1 = _pad_idx(adj1[1], EP1, M1P - 1)
    si = _pad_idx(inc1[0], EPI, 0)
    di = _pad_idx(inc1[1], EPI, M0P)
    inv0p = _pad_rows(inv0, EP0)
    inv1p = _pad_rows(inv1, EP1)
    invip = _pad_rows(inv_inc1, EPI)

    h0 = _linear(x0p, W_emb, b_emb)
    h1 = _linear(x1p, W_emb, b_emb)

    for l in range(2):
        A0, B0 = _ab(h0, Wm_adj0[l, :H], Wm_adj0[l, H:2 * H])
        A1, B1 = _ab(h1, Wm_adj1[l, :H], Wm_adj1[l, H:2 * H])
        Ai = _mm(h0, Wm_inc1[l, :H])
        Bi = _mm(h1, Wm_inc1[l, H:2 * H])
        C0 = _edgec(inv0p, Wm_adj0[l, 2 * H:], bm_adj0[l])
        C1 = _edgec(inv1p, Wm_adj1[l, 2 * H:], bm_adj1[l])
        Ci = _edgec(invip, Wm_inc1[l, 2 * H:], bm_inc1[l])

        m0 = _make_seg(HALF0, EP0, H, M0P - 1)(A0, B0, C0, s0, d0)[0]
        m1 = _make_seg(M1P // 4, EP1, H, M1P - 1, nq=2)(A1, B1, C1, s1, d1)[0]
        mi = _make_seg(HALF0, EPI, H, M0P)(Ai, Bi, Ci, si, di)[0]
        mip = jnp.pad(mi, ((0, M1P - M0P), (0, 0)))

        h0 = _update(jnp.concatenate([h0, m0], axis=1), Wu0[l], bu0[l], h0)
        h1 = _update(jnp.concatenate([h1, m1, mip], axis=1), Wu1[l], bu1[l], h1)

    g0 = _pool(h0, Wpre0[0], bpre0[0], Wpre0[1], bpre0[1], N0)
    g1 = _pool(h1, Wpre1[0], bpre1[0], Wpre1[1], bpre1[1], N1)
    out8 = _head(g0, g1, Wpost1[:H], Wpost1[H:], bpost1, Wpost2, bpost2)
    return out8[0]


# speedup vs baseline: 1.3449x; 1.0584x over previous
import functools

import jax
import jax.numpy as jnp
from jax import lax
from jax.experimental import pallas as pl
from jax.experimental.pallas import tpu as pltpu
from jax.experimental.pallas import tpu_sc as plsc

H = 128
N0, N1 = 10000, 30000
M0P, M1P = 10240, 30080
HALF0, HALF1 = 5120, 15040
EP0, EP1, EPI = 320000, 320000, 61440
BM = 320
BMC = 2560
KB = 80


def _silu(t):
    return t * jax.nn.sigmoid(t)


def _linear_body(x_ref, w_ref, b_ref, o_ref):
    o_ref[...] = jnp.dot(x_ref[...], w_ref[...],
                         preferred_element_type=jnp.float32) + b_ref[...]


def _linear(x, w, b):
    m, k = x.shape
    n = w.shape[1]
    return pl.pallas_call(
        _linear_body,
        grid=(m // BM,),
        in_specs=[pl.BlockSpec((BM, k), lambda i: (i, 0)),
                  pl.BlockSpec((k, n), lambda i: (0, 0)),
                  pl.BlockSpec((1, n), lambda i: (0, 0))],
        out_specs=pl.BlockSpec((BM, n), lambda i: (i, 0)),
        out_shape=jax.ShapeDtypeStruct((m, n), jnp.float32),
    )(x, w, b.reshape(1, n))


def _ab_body(x_ref, wa_ref, wb_ref, a_ref, b_ref):
    x = x_ref[...]
    a_ref[...] = jnp.dot(x, wa_ref[...], preferred_element_type=jnp.float32)
    b_ref[...] = jnp.dot(x, wb_ref[...], preferred_element_type=jnp.float32)


def _ab(x, wa, wb):
    m, k = x.shape
    na, nb = wa.shape[1], wb.shape[1]
    return pl.pallas_call(
        _ab_body,
        grid=(m // BM,),
        in_specs=[pl.BlockSpec((BM, k), lambda i: (i, 0)),
                  pl.BlockSpec((k, na), lambda i: (0, 0)),
                  pl.BlockSpec((k, nb), lambda i: (0, 0))],
        out_specs=[pl.BlockSpec((BM, na), lambda i: (i, 0)),
                   pl.BlockSpec((BM, nb), lambda i: (i, 0))],
        out_shape=[jax.ShapeDtypeStruct((m, na), jnp.float32),
                   jax.ShapeDtypeStruct((m, nb), jnp.float32)],
    )(x, wa, wb)


def _mm_body(x_ref, w_ref, o_ref):
    o_ref[...] = jnp.dot(x_ref[...], w_ref[...],
                         preferred_element_type=jnp.float32)


def _mm(x, w):
    m, k = x.shape
    n = w.shape[1]
    return pl.pallas_call(
        _mm_body,
        grid=(m // BM,),
        in_specs=[pl.BlockSpec((BM, k), lambda i: (i, 0)),
                  pl.BlockSpec((k, n), lambda i: (0, 0))],
        out_specs=pl.BlockSpec((BM, n), lambda i: (i, 0)),
        out_shape=jax.ShapeDtypeStruct((m, n), jnp.float32),
    )(x, w)


def _edgec_body(inv_ref, w_ref, b_ref, o_ref):
    inv = inv_ref[...]
    o_ref[...] = (b_ref[...]
                  + inv[:, 0:1] * w_ref[0:1, :]
                  + inv[:, 1:2] * w_ref[1:2, :]
                  + inv[:, 2:3] * w_ref[2:3, :])


def _edgec(inv, w, b):
    m = inv.shape[0]
    n = w.shape[1]
    return pl.pallas_call(
        _edgec_body,
        grid=(m // BMC,),
        in_specs=[pl.BlockSpec((BMC, 3), lambda i: (i, 0)),
                  pl.BlockSpec((3, n), lambda i: (0, 0)),
                  pl.BlockSpec((1, n), lambda i: (0, 0))],
        out_specs=pl.BlockSpec((BMC, n), lambda i: (i, 0)),
        out_shape=jax.ShapeDtypeStruct((m, n), jnp.float32),
    )(inv, w, b.reshape(1, n))


def _update_body(x_ref, w_ref, b_ref, r_ref, o_ref):
    t = jnp.dot(x_ref[...], w_ref[...],
                preferred_element_type=jnp.float32) + b_ref[...]
    o_ref[...] = r_ref[...] + _silu(t)


def _update(x, w, b, res):
    m, k = x.shape
    return pl.pallas_call(
        _update_body,
        grid=(m // BM,),
        in_specs=[pl.BlockSpec((BM, k), lambda i: (i, 0)),
                  pl.BlockSpec((k, H), lambda i: (0, 0)),
                  pl.BlockSpec((1, H), lambda i: (0, 0)),
                  pl.BlockSpec((BM, H), lambda i: (i, 0))],
        out_specs=pl.BlockSpec((BM, H), lambda i: (i, 0)),
        out_shape=jax.ShapeDtypeStruct((m, H), jnp.float32),
    )(x, w, b.reshape(1, H), res)


def _pool_body(x_ref, w1_ref, b1_ref, w2_ref, b2_ref, o_ref, *, n_real):
    i = pl.program_id(0)
    t = jnp.dot(x_ref[...], w1_ref[...],
                preferred_element_type=jnp.float32) + b1_ref[...]
    p = jnp.dot(_silu(t), w2_ref[...],
                preferred_element_type=jnp.float32) + b2_ref[...]
    rows = i * BM + lax.broadcasted_iota(jnp.int32, (BM, 1), 0)
    p = jnp.where(rows < n_real, p, 0.0)
    s = jnp.sum(p, axis=0, keepdims=True)

    @pl.when(i == 0)
    def _():
        o_ref[...] = jnp.zeros_like(o_ref)

    o_ref[0:1, :] = o_ref[0:1, :] + s


def _pool(x, w1, b1, w2, b2, n_real):
    m = x.shape[0]
    return pl.pallas_call(
        functools.partial(_pool_body, n_real=n_real),
        grid=(m // BM,),
        in_specs=[pl.BlockSpec((BM, H), lambda i: (i, 0)),
                  pl.BlockSpec((H, H), lambda i: (0, 0)),
                  pl.BlockSpec((1, H), lambda i: (0, 0)),
                  pl.BlockSpec((H, H), lambda i: (0, 0)),
                  pl.BlockSpec((1, H), lambda i: (0, 0))],
        out_specs=pl.BlockSpec((8, H), lambda i: (0, 0)),
        out_shape=jax.ShapeDtypeStruct((8, H), jnp.float32),
    )(x, w1, b1.reshape(1, H), w2, b2.reshape(1, H))


def _head_body(g0_ref, g1_ref, w1a_ref, w1b_ref, b1_ref, w2_ref, b2_ref, o_ref):
    t = (jnp.dot(g0_ref[...], w1a_ref[...], preferred_element_type=jnp.float32)
         + jnp.dot(g1_ref[...], w1b_ref[...], preferred_element_type=jnp.float32)
         + b1_ref[...])
    o_ref[...] = jnp.dot(_silu(t), w2_ref[...],
                         preferred_element_type=jnp.float32) + b2_ref[...]


def _head(g0, g1, w1a, w1b, b1, w2, b2):
    return pl.pallas_call(
        _head_body,
        out_shape=jax.ShapeDtypeStruct((8, H), jnp.float32),
    )(g0, g1, w1a, w1b, b1.reshape(1, H), w2, b2.reshape(1, H))


@functools.lru_cache(maxsize=None)
def _make_seg(qsize, ep, cols, dpad, nq=1):
    fch = 40
    per_tile = ep // 16
    DR = 4 * KB
    CAP = DR + 16
    CH = 400 if per_tile % 400 == 0 else 384
    assert per_tile % CH == 0 and CH % 16 == 0
    cap_out = per_tile + DR
    n_chunks = qsize // fch
    per_sub = -(-n_chunks // 16)
    acc_rows = qsize + 8
    n_out = 2 * nq * qsize
    mesh = plsc.VectorSubcoreMesh(core_axis_name="c", subcore_axis_name="s",
                                  num_cores=2, num_subcores=16)
    iota16 = lambda: lax.broadcasted_iota(jnp.int32, (16,), 0)

    vset = lambda: [pltpu.VMEM((KB,), jnp.int32),
                    pltpu.VMEM((KB,), jnp.int32),
                    pltpu.VMEM((KB,), jnp.int32),
                    pltpu.VMEM((KB,), jnp.int32),
                    pltpu.VMEM((KB, cols), jnp.float32),
                    pltpu.VMEM((KB, cols), jnp.float32),
                    pltpu.VMEM((KB, cols), jnp.float32),
                    pltpu.SemaphoreType.DMA,
                    pltpu.SemaphoreType.DMA,
                    pltpu.SemaphoreType.DMA]

    @functools.partial(
        pl.kernel,
        out_type=[jax.ShapeDtypeStruct((n_out, cols), jnp.float32),
                  jax.ShapeDtypeStruct((32 * cap_out,), jnp.int32),
                  jax.ShapeDtypeStruct((32 * cap_out,), jnp.int32),
                  jax.ShapeDtypeStruct((32 * cap_out,), jnp.int32)],
        mesh=mesh,
        compiler_params=pltpu.CompilerParams(needs_layout_passes=False),
        scratch_types=[*vset(), *vset(),
                       pltpu.VMEM((CH,), jnp.int32),
                       pltpu.VMEM((CH,), jnp.int32),
                       pltpu.VMEM((CH,), jnp.int32),
                       pltpu.VMEM((CH,), jnp.int32),
                       pltpu.SemaphoreType.DMA,
                       pltpu.SemaphoreType.DMA,
                       pltpu.VMEM((CAP,), jnp.int32),
                       pltpu.VMEM((CAP,), jnp.int32),
                       pltpu.VMEM((CAP,), jnp.int32),
                       pltpu.VMEM_SHARED((acc_rows, cols), jnp.float32)],
    )
    def seg(a_hbm, b_hbm, c_hbm, src_hbm, dst_hbm,
            out_hbm, cs_hbm, cd_hbm, ce_hbm,
            si0, di0, ei0, sl0, av0, bv0, cv0, sa0, sb0, sc0,
            si1, di1, ei1, sl1, av1, bv1, cv1, sa1, sb1, sc1,
            p1s0, p1d0, p1s1, p1d1, f0, f1, csb, cdb, ceb, acc):
        cid = lax.axis_index("c")
        sid = lax.axis_index("s")
        tile_base = sid * per_tile
        obase = (cid * 16 + sid) * cap_out
        sets = ((si0, di0, ei0, sl0, av0, bv0, cv0, sa0, sb0, sc0),
                (si1, di1, ei1, sl1, av1, bv1, cv1, sa1, sb1, sc1))

        def _zrow(i, _):
            for j in range(cols // 16):
                cv0[i, pl.ds(j * 16, 16)] = jnp.zeros((16,), jnp.float32)
            return 0
        lax.fori_loop(0, fch, _zrow, 0)

        def _drain(carry):
            ptr, tot = carry
            off = pl.multiple_of(obase + tot, 8)
            pltpu.sync_copy(csb.at[pl.ds(0, DR)], cs_hbm.at[pl.ds(off, DR)])
            pltpu.sync_copy(cdb.at[pl.ds(0, DR)], cd_hbm.at[pl.ds(off, DR)])
            pltpu.sync_copy(ceb.at[pl.ds(0, DR)], ce_hbm.at[pl.ds(off, DR)])
            csb[pl.ds(0, 16)] = csb[pl.ds(DR, 16)]
            cdb[pl.ds(0, 16)] = cdb[pl.ds(DR, 16)]
            ceb[pl.ds(0, 16)] = ceb[pl.ds(DR, 16)]
            return ptr - DR, tot + DR

        def _fstage(t, ps, pd, sem):
            eb = tile_base + t * CH
            pltpu.async_copy(src_hbm.at[pl.ds(eb, CH)], ps, sem)
            pltpu.async_copy(dst_hbm.at[pl.ds(eb, CH)], pd, sem)

        def _fwait(t, ps, pd, sem):
            eb = tile_base + t * CH
            pltpu.make_async_copy(src_hbm.at[pl.ds(eb, CH)], ps, sem).wait()
            pltpu.make_async_copy(dst_hbm.at[pl.ds(eb, CH)], pd, sem).wait()

        def _filter(base_row):
            def _do(t, ps, pd, carry):
                ebase = tile_base + t * CH

                def _grp(j, c2):
                    ptr, tot = c2
                    s16 = ps[pl.ds(j * 16, 16)]
                    d16 = pd[pl.ds(j * 16, 16)]
                    e16 = ebase + j * 16 + iota16()
                    loc = d16 - base_row
                    ok = (loc >= 0) & (loc < qsize)
                    oki = ok.astype(jnp.int32)
                    cum = plsc.cumsum(oki)
                    pos = ptr + cum - oki
                    plsc.store_scatter(csb, [pos], s16, mask=ok)
                    plsc.store_scatter(cdb, [pos], d16, mask=ok)
                    plsc.store_scatter(ceb, [pos], e16, mask=ok)
                    ptr = ptr + jnp.sum(oki)
                    return lax.cond(ptr >= DR, _drain, lambda c: c, (ptr, tot))
                return lax.fori_loop(0, CH // 16, _grp, carry)

            n2c = per_tile // CH // 2
            _fstage(0, p1s0, p1d0, f0)

            def _chunk2(g, carry):
                _fstage(2 * g + 1, p1s1, p1d1, f1)
                _fwait(2 * g, p1s0, p1d0, f0)
                carry = _do(2 * g, p1s0, p1d0, carry)

                @pl.when(g + 1 < n2c)
                def _():
                    _fstage(2 * g + 2, p1s0, p1d0, f0)
                _fwait(2 * g + 1, p1s1, p1d1, f1)
                return _do(2 * g + 1, p1s1, p1d1, carry)

            ptr, tot = lax.fori_loop(0, n2c, _chunk2, (0, 0))
            for j in range(DR // 16):
                lane = j * 16 + iota16()
                m = lane >= ptr
                sl = pl.ds(j * 16, 16)
                csb[sl] = jnp.where(m, 0, csb[sl])
                cdb[sl] = jnp.where(m, dpad, cdb[sl])
                ceb[sl] = jnp.where(m, 0, ceb[sl])
            _, tot = _drain((ptr, tot))
            return tot

        def _stage(bi, base_row, st):
            sidx, didx, eidx, slot, av, bv, cv, sa, sb, sc = st
            off = pl.multiple_of(obase + bi * KB, 8)
            pltpu.sync_copy(cs_hbm.at[pl.ds(off, KB)], sidx)
            pltpu.sync_copy(cd_hbm.at[pl.ds(off, KB)], didx)
            pltpu.sync_copy(ce_hbm.at[pl.ds(off, KB)], eidx)
            pltpu.async_copy(a_hbm.at[sidx], av, sa)
            pltpu.async_copy(b_hbm.at[didx], bv, sb)
            pltpu.async_copy(c_hbm.at[eidx], cv, sc)
            for j in range(KB // 16):
                d = didx[pl.ds(j * 16, 16)]
                loc = d - base_row
                ok = (loc >= 0) & (loc < qsize)
                slot[pl.ds(j * 16, 16)] = jnp.where(ok, loc, qsize)

        def _proc(st):
            sidx, didx, eidx, slot, av, bv, cv, sa, sb, sc = st
            pltpu.make_async_copy(a_hbm.at[sidx], av, sa).wait()
            pltpu.make_async_copy(b_hbm.at[didx], bv, sb).wait()
            pltpu.make_async_copy(c_hbm.at[eidx], cv, sc).wait()

            def _edge(i, _):
                for j in range(cols // 16):
                    sl = pl.ds(j * 16, 16)
                    t = av[i, sl] + bv[i, sl] + cv[i, sl]
                    av[i, sl] = t / (1.0 + jnp.exp(-t))
                return 0
            lax.fori_loop(0, KB, _edge, 0)
            pltpu.sync_copy(av, acc.at[slot], add=True)

        for p in range(nq):
            base_row = (2 * p + cid) * qsize

            def _zacc(t, _):
                ch = t * 16 + sid

                @pl.when(ch < n_chunks)
                def _():
                    pltpu.sync_copy(cv0.at[pl.ds(0, fch)],
                                    acc.at[pl.ds(ch * fch, fch)])
                return 0
            lax.fori_loop(0, per_sub, _zacc, 0)

            @pl.when(sid == 0)
            def _():
                pltpu.sync_copy(cv0.at[pl.ds(0, 8)], acc.at[pl.ds(qsize, 8)])

            plsc.subcore_barrier()

            total = _filter(base_row)
            n2 = total // (2 * KB)

            _stage(0, base_row, sets[0])

            def _gbody(g, _):
                _stage(2 * g + 1, base_row, sets[1])
                _proc(sets[0])

                @pl.when(g + 1 < n2)
                def _():
                    _stage(2 * g + 2, base_row, sets[0])
                _proc(sets[1])
                return 0
            lax.fori_loop(0, n2, _gbody, 0)

            plsc.subcore_barrier()

            for t in range(per_sub):
                ch = t * 16 + sid

                @pl.when(ch < n_chunks)
                def _():
                    r0 = ch * fch
                    pltpu.sync_copy(acc.at[pl.ds(r0, fch)], cv1.at[pl.ds(0, fch)])
                    pltpu.sync_copy(cv1.at[pl.ds(0, fch)],
                                    out_hbm.at[pl.ds(base_row + r0, fch)])

            if p + 1 < nq:
                plsc.subcore_barrier()
                lax.fori_loop(0, fch, _zrow, 0)

    return seg


def _pad_rows(x, m):
    return jnp.pad(x, ((0, m - x.shape[0]), (0, 0)))


def _pad_idx(idx, ep, fill):
    return jnp.pad(idx, (0, ep - idx.shape[0]), constant_values=fill)


def kernel(x0, x1, adj0, adj1, inc1, inv0, inv1, inv_inc1, W_emb, b_emb,
           Wm_adj0, bm_adj0, Wm_adj1, bm_adj1, Wm_inc1, bm_inc1, Wu0, bu0,
           Wu1, bu1, Wpre0, bpre0, Wpre1, bpre1, Wpost1, bpost1, Wpost2,
           bpost2):
    x0p = _pad_rows(x0, M0P)
    x1p = _pad_rows(x1, M1P)
    s0 = _pad_idx(adj0[0], EP0, 0)
    d0 = _pad_idx(adj0[1], EP0, M0P - 1)
    s1 = _pad_idx(adj1[0], EP1, 0)
    d1 = _pad_idx(adj1[1], EP1, M1P - 1)
    si = _pad_idx(inc1[0], EPI, 0)
    di = _pad_idx(inc1[1], EPI, M0P)
    inv0p = _pad_rows(inv0, EP0)
    inv1p = _pad_rows(inv1, EP1)
    invip = _pad_rows(inv_inc1, EPI)

    h0 = _linear(x0p, W_emb, b_emb)
    h1 = _linear(x1p, W_emb, b_emb)

    for l in range(2):
        A0, B0 = _ab(h0, Wm_adj0[l, :H], Wm_adj0[l, H:2 * H])
        A1, B1 = _ab(h1, Wm_adj1[l, :H], Wm_adj1[l, H:2 * H])
        Ai = _mm(h0, Wm_inc1[l, :H])
        Bi = _mm(h1, Wm_inc1[l, H:2 * H])
        C0 = _edgec(inv0p, Wm_adj0[l, 2 * H:], bm_adj0[l])
        C1 = _edgec(inv1p, Wm_adj1[l, 2 * H:], bm_adj1[l])
        Ci = _edgec(invip, Wm_inc1[l, 2 * H:], bm_inc1[l])

        m0 = _make_seg(HALF0, EP0, H, M0P - 1)(A0, B0, C0, s0, d0)[0]
        m1 = _make_seg(M1P // 4, EP1, H, M1P - 1, nq=2)(A1, B1, C1, s1, d1)[0]
        mi = _make_seg(HALF0, EPI, H, M0P)(Ai, Bi, Ci, si, di)[0]
        mip = jnp.pad(mi, ((0, M1P - M0P), (0, 0)))

        h0 = _update(jnp.concatenate([h0, m0], axis=1), Wu0[l], bu0[l], h0)
        h1 = _update(jnp.concatenate([h1, m1, mip], axis=1), Wu1[l], bu1[l], h1)

    g0 = _pool(h0, Wpre0[0], bpre0[0], Wpre0[1], bpre0[1], N0)
    g1 = _pool(h1, Wpre1[0], bpre1[0], Wpre1[1], bpre1[1], N1)
    out8 = _head(g0, g1, Wpost1[:H], Wpost1[H:], bpost1, Wpost2, bpost2)
    return out8[0]
